# Initial kernel scaffold; baseline (speedup 1.0000x reference)
import jax
import jax.numpy as jnp
from jax.experimental import pallas as pl


def kernel(x, gamma, beta, Wg, W1, W2):
    raise NotImplementedError("write your pallas kernel here")


# speedup vs baseline: 2.0578x; 2.0578x over previous
import functools

import jax
import jax.numpy as jnp
from jax import lax
from jax.experimental import pallas as pl
from jax.experimental.pallas import tpu as pltpu
from jax.experimental.pallas import tpu_sc as plsc

S, D, E, FF = 2048, 1024, 8, 2048
TILE = 256
TMAX = 24
P = TMAX * TILE
NW = 32
TPW = S // NW
CHUNK = 32

_SQRT1_2 = 0.7071067811865476


def _router_body(x_ref, g_ref, b_ref, wg_ref,
                 xn_ref, mask_ref, pos1_ref, pos2_ref, t1b_ref, t2b_ref,
                 sched_ref):
    x = x_ref[...]
    mu = jnp.mean(x, axis=1, keepdims=True)
    xc = x - mu
    var = jnp.mean(xc * xc, axis=1, keepdims=True)
    xn = xc * lax.rsqrt(var + 1e-5) * g_ref[...] + b_ref[...]
    xn_ref[...] = xn

    logits = jnp.dot(xn, wg_ref[...], preferred_element_type=jnp.float32)
    m = jnp.max(logits, axis=1, keepdims=True)
    ex = jnp.exp(logits - m)
    w = ex / jnp.sum(ex, axis=1, keepdims=True)

    ie = lax.broadcasted_iota(jnp.int32, (S, E), 1)
    m1 = jnp.max(w, axis=1, keepdims=True)
    i1 = jnp.min(jnp.where(w == m1, ie, E), axis=1, keepdims=True)
    wm = jnp.where(ie == i1, -1.0, w)
    m2 = jnp.max(wm, axis=1, keepdims=True)
    i2 = jnp.min(jnp.where(wm == m2, ie, E), axis=1, keepdims=True)
    ssum = m1 + m2
    t1 = m1 / ssum
    t2 = m2 / ssum
    oh1 = ie == i1
    oh2 = ie == i2
    mask_ref[...] = jnp.where(oh1, t1, 0.0) + jnp.where(oh2, t2, 0.0)
    t1b_ref[...] = jnp.broadcast_to(t1, (S, 16))
    t2b_ref[...] = jnp.broadcast_to(t2, (S, 16))

    cnt = (oh1 | oh2).astype(jnp.float32)
    csum = cnt
    k = 1
    while k < S:
        csum = csum + jnp.concatenate(
            [jnp.zeros((k, E), jnp.float32), csum[: S - k, :]], axis=0)
        k *= 2
    counts = csum[S - 1: S, :]
    rank = (csum - cnt).astype(jnp.int32)

    eye = (lax.broadcasted_iota(jnp.int32, (E, E), 0)
           == lax.broadcasted_iota(jnp.int32, (E, E), 1))
    countsT = jnp.sum(jnp.where(eye, jnp.broadcast_to(counts, (E, E)), 0.0),
                      axis=1, keepdims=True)
    pcT = ((countsT.astype(jnp.int32) + (TILE - 1)) // TILE) * TILE
    inc = pcT
    k = 1
    while k < E:
        inc = inc + jnp.concatenate(
            [jnp.zeros((k, 1), jnp.int32), inc[: E - k, :]], axis=0)
        k *= 2
    startT = inc - pcT
    start = jnp.sum(jnp.where(eye, jnp.broadcast_to(startT, (E, E)), 0),
                    axis=0, keepdims=True)

    posm = start + rank
    pos1_ref[...] = jnp.sum(jnp.where(oh1, posm, 0), axis=1, keepdims=True)
    pos2_ref[...] = jnp.sum(jnp.where(oh2, posm, 0), axis=1, keepdims=True)

    it = lax.broadcasted_iota(jnp.int32, (1, 128), 1)
    base = it * TILE
    total = jnp.sum(pcT, axis=0, keepdims=True)
    nvalid = total // TILE
    valid = base < total
    base8 = jnp.broadcast_to(base, (E, 128))
    inb = (base8 >= startT) & (base8 < startT + pcT)
    e_of = jnp.sum(jnp.where(inb, lax.broadcasted_iota(jnp.int32, (E, 128), 0), 0),
                   axis=0, keepdims=True)
    last = nvalid - 1
    r_eff = jnp.where(valid, it, last)
    e_last = jnp.sum(jnp.where(it == last, e_of, 0), axis=1, keepdims=True)
    e_eff = jnp.where(valid, e_of, e_last)
    sched_ref[...] = jnp.concatenate(
        [e_eff, r_eff, valid.astype(jnp.int32)]
        + [jnp.zeros((1, 128), jnp.int32)] * 5, axis=0)


def _router(x2, g2, b2, wgt):
    return pl.pallas_call(
        _router_body,
        out_shape=[
            jax.ShapeDtypeStruct((S, D), jnp.float32),
            jax.ShapeDtypeStruct((S, E), jnp.float32),
            jax.ShapeDtypeStruct((S, 1), jnp.int32),
            jax.ShapeDtypeStruct((S, 1), jnp.int32),
            jax.ShapeDtypeStruct((S, 16), jnp.float32),
            jax.ShapeDtypeStruct((S, 16), jnp.float32),
            jax.ShapeDtypeStruct((8, 128), jnp.int32),
        ],
    )(x2, g2, b2, wgt)


def _gmm_body(e_ref, r_ref, v_ref, xs_ref, w1_ref, w2_ref, y_ref):
    i = pl.program_id(0)

    @pl.when(v_ref[i] == 1)
    def _():
        xb = xs_ref[...].astype(jnp.bfloat16)
        h = jnp.dot(xb, w1_ref[0], preferred_element_type=jnp.float32)
        xp = h[:, :FF]
        gt = h[:, FF:]
        og = xp * (0.5 * gt * (1.0 + lax.erf(gt * _SQRT1_2)))
        y_ref[...] = jnp.dot(og.astype(jnp.bfloat16), w2_ref[0],
                             preferred_element_type=jnp.float32)


def _gmm(e_eff, r_eff, valid, xs, w1t, w2t):
    grid_spec = pltpu.PrefetchScalarGridSpec(
        num_scalar_prefetch=3,
        grid=(TMAX,),
        in_specs=[
            pl.BlockSpec((TILE, D), lambda i, e, r, v: (r[i], 0)),
            pl.BlockSpec((1, D, 2 * FF), lambda i, e, r, v: (e[i], 0, 0)),
            pl.BlockSpec((1, FF, D), lambda i, e, r, v: (e[i], 0, 0)),
        ],
        out_specs=pl.BlockSpec((TILE, D), lambda i, e, r, v: (r[i], 0)),
    )
    return pl.pallas_call(
        _gmm_body,
        grid_spec=grid_spec,
        out_shape=jax.ShapeDtypeStruct((P, D), jnp.float32),
        compiler_params=pltpu.CompilerParams(
            dimension_semantics=("arbitrary",)),
    )(e_eff, r_eff, valid, xs, w1t, w2t)


def _sc_scatter(xn, pos1r, pos2r):
    mesh = plsc.VectorSubcoreMesh(core_axis_name="c", subcore_axis_name="s")

    @functools.partial(
        pl.kernel, mesh=mesh,
        out_type=jax.ShapeDtypeStruct((P, D), jnp.float32),
        scratch_types=[
            pltpu.VMEM((TPW,), jnp.int32),
            pltpu.VMEM((TPW,), jnp.int32),
            pltpu.VMEM((TPW, D), jnp.float32),
            pltpu.SemaphoreType.DMA,
            pltpu.SemaphoreType.DMA,
        ],
    )
    def k(xn_hbm, p1_hbm, p2_hbm, xs_hbm, i1_v, i2_v, rows_v, sem1, sem2):
        wid = lax.axis_index("s") * 2 + lax.axis_index("c")
        base = wid * TPW
        pltpu.sync_copy(p1_hbm.at[wid], i1_v)
        pltpu.sync_copy(p2_hbm.at[wid], i2_v)
        pltpu.sync_copy(xn_hbm.at[pl.ds(base, TPW)], rows_v)
        c1 = pltpu.async_copy(rows_v, xs_hbm.at[i1_v], sem1)
        c2 = pltpu.async_copy(rows_v, xs_hbm.at[i2_v], sem2)
        c1.wait()
        c2.wait()

    return k(xn, pos1r, pos2r)


def _sc_combine(x2, ys, p1c, p2c, t1c, t2c):
    mesh = plsc.VectorSubcoreMesh(core_axis_name="c", subcore_axis_name="s")
    nchunk = TPW // CHUNK

    @functools.partial(
        pl.kernel, mesh=mesh,
        out_type=jax.ShapeDtypeStruct((S, D), jnp.float32),
        scratch_types=[
            pltpu.VMEM((CHUNK,), jnp.int32),
            pltpu.VMEM((CHUNK,), jnp.int32),
            pltpu.VMEM((CHUNK * 16,), jnp.float32),
            pltpu.VMEM((CHUNK * 16,), jnp.float32),
            pltpu.VMEM((CHUNK, D), jnp.float32),
            pltpu.VMEM((CHUNK, D), jnp.float32),
            pltpu.VMEM((CHUNK, D), jnp.float32),
            pltpu.SemaphoreType.DMA,
            pltpu.SemaphoreType.DMA,
        ],
    )
    def k(x_hbm, ys_hbm, p1_hbm, p2_hbm, t1_hbm, t2_hbm, out_hbm,
          i1_v, i2_v, t1_v, t2_v, acc_v, g1_v, g2_v, sem1, sem2):
        wid = lax.axis_index("s") * 2 + lax.axis_index("c")
        for c in range(nchunk):
            base = wid * TPW + c * CHUNK
            pltpu.sync_copy(p1_hbm.at[wid, c], i1_v)
            pltpu.sync_copy(p2_hbm.at[wid, c], i2_v)
            pltpu.sync_copy(t1_hbm.at[wid, c], t1_v)
            pltpu.sync_copy(t2_hbm.at[wid, c], t2_v)
            cp1 = pltpu.async_copy(ys_hbm.at[i1_v], g1_v, sem1)
            cp2 = pltpu.async_copy(ys_hbm.at[i2_v], g2_v, sem2)
            pltpu.sync_copy(x_hbm.at[pl.ds(base, CHUNK)], acc_v)
            cp1.wait()
            cp2.wait()

            def body(r, carry):
                t1s = t1_v[pl.ds(r * 16, 16)]
                t2s = t2_v[pl.ds(r * 16, 16)]
                for g in range(D // 16):
                    sl = pl.ds(g * 16, 16)
                    acc_v[r, sl] = (acc_v[r, sl]
                                    + t1s * g1_v[r, sl] + t2s * g2_v[r, sl])
                return carry

            lax.fori_loop(0, CHUNK, body, 0)
            pltpu.sync_copy(acc_v, out_hbm.at[pl.ds(base, CHUNK)])

    return k(x2, ys, p1c, p2c, t1c, t2c)


def kernel(x, gamma, beta, Wg, W1, W2):
    x2 = x.reshape(S, D)
    g2 = gamma.reshape(1, D)
    b2 = beta.reshape(1, D)
    wgt = Wg.T

    xn, mask, pos1, pos2, t1, t2, sched = _router(x2, g2, b2, wgt)
    e_eff = sched[0, :TMAX]
    r_eff = sched[1, :TMAX]
    valid = sched[2, :TMAX]

    xs = _sc_scatter(xn, pos1.reshape(NW, TPW), pos2.reshape(NW, TPW))

    w1t = jnp.transpose(W1.astype(jnp.bfloat16), (0, 2, 1))
    w2t = jnp.transpose(W2.astype(jnp.bfloat16), (0, 2, 1))
    ys = _gmm(e_eff, r_eff, valid, xs, w1t, w2t)

    nch = TPW // CHUNK
    out2 = _sc_combine(
        x2, ys,
        pos1.reshape(NW, nch, CHUNK), pos2.reshape(NW, nch, CHUNK),
        t1.reshape(NW, nch, CHUNK * 16), t2.reshape(NW, nch, CHUNK * 16))
    return out2.reshape(1, S, D), mask.reshape(1, S, E)


# speedup vs baseline: 2.4922x; 1.2111x over previous
import functools

import jax
import jax.numpy as jnp
from jax import lax
from jax.experimental import pallas as pl
from jax.experimental.pallas import tpu as pltpu
from jax.experimental.pallas import tpu_sc as plsc

S, D, E, FF = 2048, 1024, 8, 2048
TILE = 256
TMAX = 24
P = TMAX * TILE
NW = 32
TPW = S // NW
CHUNK = 32

_SQRT1_2 = 0.7071067811865476


def _router_body(x_ref, g_ref, b_ref, wg_ref,
                 xn_ref, mask_ref, pos1_ref, pos2_ref, t1b_ref, t2b_ref,
                 sched_ref):
    x = x_ref[...]
    mu = jnp.mean(x, axis=1, keepdims=True)
    xc = x - mu
    var = jnp.mean(xc * xc, axis=1, keepdims=True)
    xn = xc * lax.rsqrt(var + 1e-5) * g_ref[...] + b_ref[...]
    xn_ref[...] = xn

    logits = jnp.dot(xn, wg_ref[...], preferred_element_type=jnp.float32)
    m = jnp.max(logits, axis=1, keepdims=True)
    ex = jnp.exp(logits - m)
    w = ex / jnp.sum(ex, axis=1, keepdims=True)

    ie = lax.broadcasted_iota(jnp.int32, (S, E), 1)
    m1 = jnp.max(w, axis=1, keepdims=True)
    i1 = jnp.min(jnp.where(w == m1, ie, E), axis=1, keepdims=True)
    wm = jnp.where(ie == i1, -1.0, w)
    m2 = jnp.max(wm, axis=1, keepdims=True)
    i2 = jnp.min(jnp.where(wm == m2, ie, E), axis=1, keepdims=True)
    ssum = m1 + m2
    t1 = m1 / ssum
    t2 = m2 / ssum
    oh1 = ie == i1
    oh2 = ie == i2
    mask_ref[...] = jnp.where(oh1, t1, 0.0) + jnp.where(oh2, t2, 0.0)
    t1b_ref[...] = jnp.broadcast_to(t1, (S, 16))
    t2b_ref[...] = jnp.broadcast_to(t2, (S, 16))

    cnt = (oh1 | oh2).astype(jnp.float32)
    csum = cnt
    k = 1
    while k < S:
        csum = csum + jnp.concatenate(
            [jnp.zeros((k, E), jnp.float32), csum[: S - k, :]], axis=0)
        k *= 2
    counts = csum[S - 1: S, :]
    rank = (csum - cnt).astype(jnp.int32)

    eye = (lax.broadcasted_iota(jnp.int32, (E, E), 0)
           == lax.broadcasted_iota(jnp.int32, (E, E), 1))
    countsT = jnp.sum(jnp.where(eye, jnp.broadcast_to(counts, (E, E)), 0.0),
                      axis=1, keepdims=True)
    pcT = ((countsT.astype(jnp.int32) + (TILE - 1)) // TILE) * TILE
    inc = pcT
    k = 1
    while k < E:
        inc = inc + jnp.concatenate(
            [jnp.zeros((k, 1), jnp.int32), inc[: E - k, :]], axis=0)
        k *= 2
    startT = inc - pcT
    start = jnp.sum(jnp.where(eye, jnp.broadcast_to(startT, (E, E)), 0),
                    axis=0, keepdims=True)

    posm = start + rank
    pos1_ref[...] = jnp.sum(jnp.where(oh1, posm, 0), axis=1, keepdims=True)
    pos2_ref[...] = jnp.sum(jnp.where(oh2, posm, 0), axis=1, keepdims=True)

    it = lax.broadcasted_iota(jnp.int32, (1, 128), 1)
    base = it * TILE
    total = jnp.sum(pcT, axis=0, keepdims=True)
    nvalid = total // TILE
    valid = base < total
    base8 = jnp.broadcast_to(base, (E, 128))
    inb = (base8 >= startT) & (base8 < startT + pcT)
    e_of = jnp.sum(jnp.where(inb, lax.broadcasted_iota(jnp.int32, (E, 128), 0), 0),
                   axis=0, keepdims=True)
    last = nvalid - 1
    r_eff = jnp.where(valid, it, last)
    e_last = jnp.sum(jnp.where(it == last, e_of, 0), axis=1, keepdims=True)
    e_eff = jnp.where(valid, e_of, e_last)
    sched_ref[...] = jnp.concatenate(
        [e_eff, r_eff, valid.astype(jnp.int32)]
        + [jnp.zeros((1, 128), jnp.int32)] * 5, axis=0)


def _router(x2, g2, b2, wgt):
    return pl.pallas_call(
        _router_body,
        out_shape=[
            jax.ShapeDtypeStruct((S, D), jnp.float32),
            jax.ShapeDtypeStruct((S, E), jnp.float32),
            jax.ShapeDtypeStruct((S, 1), jnp.int32),
            jax.ShapeDtypeStruct((S, 1), jnp.int32),
            jax.ShapeDtypeStruct((S, 16), jnp.float32),
            jax.ShapeDtypeStruct((S, 16), jnp.float32),
            jax.ShapeDtypeStruct((8, 128), jnp.int32),
        ],
    )(x2, g2, b2, wgt)


def _gmm_body(e_ref, r_ref, v_ref, xs_ref, w1_ref, w2_ref, y_ref):
    i = pl.program_id(0)

    @pl.when(v_ref[i] == 1)
    def _():
        xb = xs_ref[...].astype(jnp.bfloat16)
        h = lax.dot_general(xb, w1_ref[0], (((1,), (1,)), ((), ())),
                            preferred_element_type=jnp.float32)
        xp = h[:, :FF]
        gt = h[:, FF:]
        og = xp * (0.5 * gt * (1.0 + lax.erf(gt * _SQRT1_2)))
        y_ref[...] = lax.dot_general(og.astype(jnp.bfloat16), w2_ref[0],
                                     (((1,), (1,)), ((), ())),
                                     preferred_element_type=jnp.float32)


def _gmm(e_eff, r_eff, valid, xs, w1t, w2t):
    grid_spec = pltpu.PrefetchScalarGridSpec(
        num_scalar_prefetch=3,
        grid=(TMAX,),
        in_specs=[
            pl.BlockSpec((TILE, D), lambda i, e, r, v: (r[i], 0)),
            pl.BlockSpec((1, 2 * FF, D), lambda i, e, r, v: (e[i], 0, 0)),
            pl.BlockSpec((1, D, FF), lambda i, e, r, v: (e[i], 0, 0)),
        ],
        out_specs=pl.BlockSpec((TILE, D), lambda i, e, r, v: (r[i], 0)),
    )
    return pl.pallas_call(
        _gmm_body,
        grid_spec=grid_spec,
        out_shape=jax.ShapeDtypeStruct((P, D), jnp.float32),
        compiler_params=pltpu.CompilerParams(
            dimension_semantics=("arbitrary",)),
    )(e_eff, r_eff, valid, xs, w1t, w2t)


def _sc_scatter(xn, pos1r, pos2r):
    mesh = plsc.VectorSubcoreMesh(core_axis_name="c", subcore_axis_name="s")

    @functools.partial(
        pl.kernel, mesh=mesh,
        out_type=jax.ShapeDtypeStruct((P, D), jnp.float32),
        scratch_types=[
            pltpu.VMEM((TPW,), jnp.int32),
            pltpu.VMEM((TPW,), jnp.int32),
            pltpu.VMEM((TPW, D), jnp.float32),
            pltpu.SemaphoreType.DMA,
            pltpu.SemaphoreType.DMA,
        ],
    )
    def k(xn_hbm, p1_hbm, p2_hbm, xs_hbm, i1_v, i2_v, rows_v, sem1, sem2):
        wid = lax.axis_index("s") * 2 + lax.axis_index("c")
        base = wid * TPW
        pltpu.sync_copy(p1_hbm.at[wid], i1_v)
        pltpu.sync_copy(p2_hbm.at[wid], i2_v)
        pltpu.sync_copy(xn_hbm.at[pl.ds(base, TPW)], rows_v)
        c1 = pltpu.async_copy(rows_v, xs_hbm.at[i1_v], sem1)
        c2 = pltpu.async_copy(rows_v, xs_hbm.at[i2_v], sem2)
        c1.wait()
        c2.wait()

    return k(xn, pos1r, pos2r)


def _sc_combine(x2, ys, p1c, p2c, t1c, t2c):
    mesh = plsc.VectorSubcoreMesh(core_axis_name="c", subcore_axis_name="s")
    nchunk = TPW // CHUNK

    @functools.partial(
        pl.kernel, mesh=mesh,
        out_type=jax.ShapeDtypeStruct((S, D), jnp.float32),
        scratch_types=[
            pltpu.VMEM((CHUNK,), jnp.int32),
            pltpu.VMEM((CHUNK,), jnp.int32),
            pltpu.VMEM((CHUNK * 16,), jnp.float32),
            pltpu.VMEM((CHUNK * 16,), jnp.float32),
            pltpu.VMEM((CHUNK, D), jnp.float32),
            pltpu.VMEM((CHUNK, D), jnp.float32),
            pltpu.VMEM((CHUNK, D), jnp.float32),
            pltpu.SemaphoreType.DMA,
            pltpu.SemaphoreType.DMA,
        ],
    )
    def k(x_hbm, ys_hbm, p1_hbm, p2_hbm, t1_hbm, t2_hbm, out_hbm,
          i1_v, i2_v, t1_v, t2_v, acc_v, g1_v, g2_v, sem1, sem2):
        wid = lax.axis_index("s") * 2 + lax.axis_index("c")
        for c in range(nchunk):
            base = wid * TPW + c * CHUNK
            pltpu.sync_copy(p1_hbm.at[wid, c], i1_v)
            pltpu.sync_copy(p2_hbm.at[wid, c], i2_v)
            pltpu.sync_copy(t1_hbm.at[wid, c], t1_v)
            pltpu.sync_copy(t2_hbm.at[wid, c], t2_v)
            cp1 = pltpu.async_copy(ys_hbm.at[i1_v], g1_v, sem1)
            cp2 = pltpu.async_copy(ys_hbm.at[i2_v], g2_v, sem2)
            pltpu.sync_copy(x_hbm.at[pl.ds(base, CHUNK)], acc_v)
            cp1.wait()
            cp2.wait()

            def body(r, carry):
                t1s = t1_v[pl.ds(r * 16, 16)]
                t2s = t2_v[pl.ds(r * 16, 16)]
                for g in range(D // 16):
                    sl = pl.ds(g * 16, 16)
                    acc_v[r, sl] = (acc_v[r, sl]
                                    + t1s * g1_v[r, sl] + t2s * g2_v[r, sl])
                return carry

            lax.fori_loop(0, CHUNK, body, 0)
            pltpu.sync_copy(acc_v, out_hbm.at[pl.ds(base, CHUNK)])

    return k(x2, ys, p1c, p2c, t1c, t2c)


def kernel(x, gamma, beta, Wg, W1, W2):
    x2 = x.reshape(S, D)
    g2 = gamma.reshape(1, D)
    b2 = beta.reshape(1, D)
    wgt = Wg.T

    xn, mask, pos1, pos2, t1, t2, sched = _router(x2, g2, b2, wgt)
    e_eff = sched[0, :TMAX]
    r_eff = sched[1, :TMAX]
    valid = sched[2, :TMAX]

    xs = _sc_scatter(xn, pos1.reshape(NW, TPW), pos2.reshape(NW, TPW))

    w1b = W1.astype(jnp.bfloat16)
    w2b = W2.astype(jnp.bfloat16)
    ys = _gmm(e_eff, r_eff, valid, xs, w1b, w2b)

    nch = TPW // CHUNK
    out2 = _sc_combine(
        x2, ys,
        pos1.reshape(NW, nch, CHUNK), pos2.reshape(NW, nch, CHUNK),
        t1.reshape(NW, nch, CHUNK * 16), t2.reshape(NW, nch, CHUNK * 16))
    return out2.reshape(1, S, D), mask.reshape(1, S, E)


# speedup vs baseline: 2.5679x; 1.0304x over previous
import functools

import jax
import jax.numpy as jnp
from jax import lax
from jax.experimental import pallas as pl
from jax.experimental.pallas import tpu as pltpu
from jax.experimental.pallas import tpu_sc as plsc

S, D, E, FF = 2048, 1024, 8, 2048
TILE = 256
TMAX = 24
P = TMAX * TILE
NW = 32
TPW = S // NW
CHUNK = 32

_SQRT1_2 = 0.7071067811865476


def _router_body(x_ref, g_ref, b_ref, wg_ref,
                 xn_ref, mask_ref, pos1_ref, pos2_ref, t1b_ref, t2b_ref,
                 sched_ref):
    x = x_ref[...]
    mu = jnp.mean(x, axis=1, keepdims=True)
    xc = x - mu
    var = jnp.mean(xc * xc, axis=1, keepdims=True)
    xn = xc * lax.rsqrt(var + 1e-5) * g_ref[...] + b_ref[...]
    xn_ref[...] = xn

    logits = jnp.dot(xn, wg_ref[...], preferred_element_type=jnp.float32)
    m = jnp.max(logits, axis=1, keepdims=True)
    ex = jnp.exp(logits - m)
    w = ex / jnp.sum(ex, axis=1, keepdims=True)

    ie = lax.broadcasted_iota(jnp.int32, (S, E), 1)
    m1 = jnp.max(w, axis=1, keepdims=True)
    i1 = jnp.min(jnp.where(w == m1, ie, E), axis=1, keepdims=True)
    wm = jnp.where(ie == i1, -1.0, w)
    m2 = jnp.max(wm, axis=1, keepdims=True)
    i2 = jnp.min(jnp.where(wm == m2, ie, E), axis=1, keepdims=True)
    ssum = m1 + m2
    t1 = m1 / ssum
    t2 = m2 / ssum
    oh1 = ie == i1
    oh2 = ie == i2
    mask_ref[...] = jnp.where(oh1, t1, 0.0) + jnp.where(oh2, t2, 0.0)
    nrows = S // CHUNK
    t1b_ref[...] = jnp.broadcast_to(
        t1.reshape(nrows, CHUNK)[:, :, None], (nrows, CHUNK, 16)
    ).reshape(nrows, CHUNK * 16)
    t2b_ref[...] = jnp.broadcast_to(
        t2.reshape(nrows, CHUNK)[:, :, None], (nrows, CHUNK, 16)
    ).reshape(nrows, CHUNK * 16)

    cnt = (oh1 | oh2).astype(jnp.float32)
    csum = cnt
    k = 1
    while k < S:
        csum = csum + jnp.concatenate(
            [jnp.zeros((k, E), jnp.float32), csum[: S - k, :]], axis=0)
        k *= 2
    counts = csum[S - 1: S, :]
    rank = (csum - cnt).astype(jnp.int32)

    eye = (lax.broadcasted_iota(jnp.int32, (E, E), 0)
           == lax.broadcasted_iota(jnp.int32, (E, E), 1))
    countsT = jnp.sum(jnp.where(eye, jnp.broadcast_to(counts, (E, E)), 0.0),
                      axis=1, keepdims=True)
    pcT = ((countsT.astype(jnp.int32) + (TILE - 1)) // TILE) * TILE
    inc = pcT
    k = 1
    while k < E:
        inc = inc + jnp.concatenate(
            [jnp.zeros((k, 1), jnp.int32), inc[: E - k, :]], axis=0)
        k *= 2
    startT = inc - pcT
    start = jnp.sum(jnp.where(eye, jnp.broadcast_to(startT, (E, E)), 0),
                    axis=0, keepdims=True)

    posm = start + rank
    pos1 = jnp.sum(jnp.where(oh1, posm, 0), axis=1, keepdims=True)
    pos2 = jnp.sum(jnp.where(oh2, posm, 0), axis=1, keepdims=True)
    pos1_ref[...] = pos1.reshape(S // 128, 128)
    pos2_ref[...] = pos2.reshape(S // 128, 128)

    it = lax.broadcasted_iota(jnp.int32, (1, 128), 1)
    base = it * TILE
    total = jnp.sum(pcT, axis=0, keepdims=True)
    nvalid = total // TILE
    valid = base < total
    base8 = jnp.broadcast_to(base, (E, 128))
    inb = (base8 >= startT) & (base8 < startT + pcT)
    e_of = jnp.sum(jnp.where(inb, lax.broadcasted_iota(jnp.int32, (E, 128), 0), 0),
                   axis=0, keepdims=True)
    last = nvalid - 1
    r_eff = jnp.where(valid, it, last)
    e_last = jnp.sum(jnp.where(it == last, e_of, 0), axis=1, keepdims=True)
    e_eff = jnp.where(valid, e_of, e_last)
    sched_ref[...] = jnp.concatenate(
        [e_eff, r_eff, valid.astype(jnp.int32)]
        + [jnp.zeros((1, 128), jnp.int32)] * 5, axis=0)


def _router(x2, g2, b2, wgt):
    return pl.pallas_call(
        _router_body,
        out_shape=[
            jax.ShapeDtypeStruct((S, D), jnp.float32),
            jax.ShapeDtypeStruct((S, E), jnp.float32),
            jax.ShapeDtypeStruct((S // 128, 128), jnp.int32),
            jax.ShapeDtypeStruct((S // 128, 128), jnp.int32),
            jax.ShapeDtypeStruct((S // CHUNK, CHUNK * 16), jnp.float32),
            jax.ShapeDtypeStruct((S // CHUNK, CHUNK * 16), jnp.float32),
            jax.ShapeDtypeStruct((8, 128), jnp.int32),
        ],
    )(x2, g2, b2, wgt)


def _gmm_body(s_ref, xs_ref, w1_ref, w2_ref, y_ref):
    i = pl.program_id(0)

    @pl.when(s_ref[2, i] == 1)
    def _():
        xb = xs_ref[...].astype(jnp.bfloat16)
        h = lax.dot_general(xb, w1_ref[0], (((1,), (1,)), ((), ())),
                            preferred_element_type=jnp.float32)
        xp = h[:, :FF]
        gt = h[:, FF:]
        og = xp * (0.5 * gt * (1.0 + lax.erf(gt * _SQRT1_2)))
        y_ref[...] = lax.dot_general(og.astype(jnp.bfloat16), w2_ref[0],
                                     (((1,), (1,)), ((), ())),
                                     preferred_element_type=jnp.float32)


def _gmm(sched, xs, w1t, w2t):
    grid_spec = pltpu.PrefetchScalarGridSpec(
        num_scalar_prefetch=1,
        grid=(TMAX,),
        in_specs=[
            pl.BlockSpec((TILE, D), lambda i, s: (s[1, i], 0)),
            pl.BlockSpec((1, 2 * FF, D), lambda i, s: (s[0, i], 0, 0)),
            pl.BlockSpec((1, D, FF), lambda i, s: (s[0, i], 0, 0)),
        ],
        out_specs=pl.BlockSpec((TILE, D), lambda i, s: (s[1, i], 0)),
    )
    return pl.pallas_call(
        _gmm_body,
        grid_spec=grid_spec,
        out_shape=jax.ShapeDtypeStruct((P, D), jnp.float32),
        compiler_params=pltpu.CompilerParams(
            dimension_semantics=("arbitrary",)),
    )(sched, xs, w1t, w2t)


def _sc_scatter(xn, pos1r, pos2r):
    mesh = plsc.VectorSubcoreMesh(core_axis_name="c", subcore_axis_name="s")

    @functools.partial(
        pl.kernel, mesh=mesh,
        out_type=jax.ShapeDtypeStruct((P, D), jnp.float32),
        scratch_types=[
            pltpu.VMEM((TPW,), jnp.int32),
            pltpu.VMEM((TPW,), jnp.int32),
            pltpu.VMEM((TPW, D), jnp.float32),
            pltpu.SemaphoreType.DMA,
            pltpu.SemaphoreType.DMA,
        ],
    )
    def k(xn_hbm, p1_hbm, p2_hbm, xs_hbm, i1_v, i2_v, rows_v, sem1, sem2):
        wid = lax.axis_index("s") * 2 + lax.axis_index("c")
        base = wid * TPW
        row = wid // 2
        lane = (wid % 2) * TPW
        pltpu.sync_copy(p1_hbm.at[row, pl.ds(lane, TPW)], i1_v)
        pltpu.sync_copy(p2_hbm.at[row, pl.ds(lane, TPW)], i2_v)
        pltpu.sync_copy(xn_hbm.at[pl.ds(base, TPW)], rows_v)
        c1 = pltpu.async_copy(rows_v, xs_hbm.at[i1_v], sem1)
        c2 = pltpu.async_copy(rows_v, xs_hbm.at[i2_v], sem2)
        c1.wait()
        c2.wait()

    return k(xn, pos1r, pos2r)


def _sc_combine(x2, ys, p1c, p2c, t1c, t2c):
    mesh = plsc.VectorSubcoreMesh(core_axis_name="c", subcore_axis_name="s")
    nchunk = TPW // CHUNK

    @functools.partial(
        pl.kernel, mesh=mesh,
        out_type=jax.ShapeDtypeStruct((S, D), jnp.float32),
        scratch_types=[
            pltpu.VMEM((CHUNK,), jnp.int32),
            pltpu.VMEM((CHUNK,), jnp.int32),
            pltpu.VMEM((CHUNK * 16,), jnp.float32),
            pltpu.VMEM((CHUNK * 16,), jnp.float32),
            pltpu.VMEM((CHUNK, D), jnp.float32),
            pltpu.VMEM((CHUNK, D), jnp.float32),
            pltpu.VMEM((CHUNK, D), jnp.float32),
            pltpu.SemaphoreType.DMA,
            pltpu.SemaphoreType.DMA,
        ],
    )
    def k(x_hbm, ys_hbm, p1_hbm, p2_hbm, t1_hbm, t2_hbm, out_hbm,
          i1_v, i2_v, t1_v, t2_v, acc_v, g1_v, g2_v, sem1, sem2):
        wid = lax.axis_index("s") * 2 + lax.axis_index("c")
        for c in range(nchunk):
            base = wid * TPW + c * CHUNK
            prow = wid // 2
            plane = (wid % 2) * TPW + c * CHUNK
            pltpu.sync_copy(p1_hbm.at[prow, pl.ds(plane, CHUNK)], i1_v)
            pltpu.sync_copy(p2_hbm.at[prow, pl.ds(plane, CHUNK)], i2_v)
            pltpu.sync_copy(t1_hbm.at[wid * nchunk + c], t1_v)
            pltpu.sync_copy(t2_hbm.at[wid * nchunk + c], t2_v)
            cp1 = pltpu.async_copy(ys_hbm.at[i1_v], g1_v, sem1)
            cp2 = pltpu.async_copy(ys_hbm.at[i2_v], g2_v, sem2)
            pltpu.sync_copy(x_hbm.at[pl.ds(base, CHUNK)], acc_v)
            cp1.wait()
            cp2.wait()

            def body(r, carry):
                t1s = t1_v[pl.ds(r * 16, 16)]
                t2s = t2_v[pl.ds(r * 16, 16)]
                for g in range(D // 16):
                    sl = pl.ds(g * 16, 16)
                    acc_v[r, sl] = (acc_v[r, sl]
                                    + t1s * g1_v[r, sl] + t2s * g2_v[r, sl])
                return carry

            lax.fori_loop(0, CHUNK, body, 0)
            pltpu.sync_copy(acc_v, out_hbm.at[pl.ds(base, CHUNK)])

    return k(x2, ys, p1c, p2c, t1c, t2c)


def kernel(x, gamma, beta, Wg, W1, W2):
    x2 = x.reshape(S, D)
    g2 = gamma.reshape(1, D)
    b2 = beta.reshape(1, D)
    wgt = Wg.T

    w1b = W1.astype(jnp.bfloat16)
    w2b = W2.astype(jnp.bfloat16)

    xn, mask, pos1, pos2, t1b, t2b, sched = _router(x2, g2, b2, wgt)
    xs = _sc_scatter(xn, pos1, pos2)
    ys = _gmm(sched, xs, w1b, w2b)
    out2 = _sc_combine(x2, ys, pos1, pos2, t1b, t2b)
    return out2.reshape(1, S, D), mask.reshape(1, S, E)


# speedup vs baseline: 2.7608x; 1.0751x over previous
import functools

import jax
import jax.numpy as jnp
from jax import lax
from jax.experimental import pallas as pl
from jax.experimental.pallas import tpu as pltpu
from jax.experimental.pallas import tpu_sc as plsc

S, D, E, FF = 2048, 1024, 8, 2048
TILE = 256
TMAX = 24
P = TMAX * TILE
NW = 32
TPW = S // NW
CHUNK = 32

_SQRT1_2 = 0.7071067811865476


def _router_body(x_ref, g_ref, b_ref, wg_ref,
                 xn_ref, mask_ref, pos1_ref, pos2_ref, t1b_ref, t2b_ref,
                 sched_ref):
    x = x_ref[...]
    mu = jnp.mean(x, axis=1, keepdims=True)
    xc = x - mu
    var = jnp.mean(xc * xc, axis=1, keepdims=True)
    xn = xc * lax.rsqrt(var + 1e-5) * g_ref[...] + b_ref[...]
    xn_ref[...] = xn

    logits = jnp.dot(xn, wg_ref[...], preferred_element_type=jnp.float32)
    m = jnp.max(logits, axis=1, keepdims=True)
    ex = jnp.exp(logits - m)
    w = ex / jnp.sum(ex, axis=1, keepdims=True)

    ie = lax.broadcasted_iota(jnp.int32, (S, E), 1)
    m1 = jnp.max(w, axis=1, keepdims=True)
    i1 = jnp.min(jnp.where(w == m1, ie, E), axis=1, keepdims=True)
    wm = jnp.where(ie == i1, -1.0, w)
    m2 = jnp.max(wm, axis=1, keepdims=True)
    i2 = jnp.min(jnp.where(wm == m2, ie, E), axis=1, keepdims=True)
    ssum = m1 + m2
    t1 = m1 / ssum
    t2 = m2 / ssum
    oh1 = ie == i1
    oh2 = ie == i2
    mask_ref[...] = jnp.where(oh1, t1, 0.0) + jnp.where(oh2, t2, 0.0)
    nrows = S // CHUNK
    t1b_ref[...] = jnp.broadcast_to(
        t1.reshape(nrows, CHUNK)[:, :, None], (nrows, CHUNK, 16)
    ).reshape(nrows, CHUNK * 16)
    t2b_ref[...] = jnp.broadcast_to(
        t2.reshape(nrows, CHUNK)[:, :, None], (nrows, CHUNK, 16)
    ).reshape(nrows, CHUNK * 16)

    cnt = (oh1 | oh2).astype(jnp.float32)
    csum = cnt
    k = 1
    while k < S:
        csum = csum + jnp.concatenate(
            [jnp.zeros((k, E), jnp.float32), csum[: S - k, :]], axis=0)
        k *= 2
    counts = csum[S - 1: S, :]
    rank = (csum - cnt).astype(jnp.int32)

    eye = (lax.broadcasted_iota(jnp.int32, (E, E), 0)
           == lax.broadcasted_iota(jnp.int32, (E, E), 1))
    countsT = jnp.sum(jnp.where(eye, jnp.broadcast_to(counts, (E, E)), 0.0),
                      axis=1, keepdims=True)
    pcT = ((countsT.astype(jnp.int32) + (TILE - 1)) // TILE) * TILE
    inc = pcT
    k = 1
    while k < E:
        inc = inc + jnp.concatenate(
            [jnp.zeros((k, 1), jnp.int32), inc[: E - k, :]], axis=0)
        k *= 2
    startT = inc - pcT
    start = jnp.sum(jnp.where(eye, jnp.broadcast_to(startT, (E, E)), 0),
                    axis=0, keepdims=True)

    posm = start + rank
    pos1 = jnp.sum(jnp.where(oh1, posm, 0), axis=1, keepdims=True)
    pos2 = jnp.sum(jnp.where(oh2, posm, 0), axis=1, keepdims=True)
    pos1_ref[...] = pos1.reshape(S // 128, 128)
    pos2_ref[...] = pos2.reshape(S // 128, 128)

    it = lax.broadcasted_iota(jnp.int32, (1, 128), 1)
    base = it * TILE
    total = jnp.sum(pcT, axis=0, keepdims=True)
    nvalid = total // TILE
    valid = base < total
    base8 = jnp.broadcast_to(base, (E, 128))
    inb = (base8 >= startT) & (base8 < startT + pcT)
    e_of = jnp.sum(jnp.where(inb, lax.broadcasted_iota(jnp.int32, (E, 128), 0), 0),
                   axis=0, keepdims=True)
    last = nvalid - 1
    r_eff = jnp.where(valid, it, last)
    e_last = jnp.sum(jnp.where(it == last, e_of, 0), axis=1, keepdims=True)
    e_eff = jnp.where(valid, e_of, e_last)
    sched_ref[...] = jnp.concatenate(
        [e_eff, r_eff, valid.astype(jnp.int32)]
        + [jnp.zeros((1, 128), jnp.int32)] * 5, axis=0)


def _router(x2, g2, b2, wgt):
    return pl.pallas_call(
        _router_body,
        out_shape=[
            jax.ShapeDtypeStruct((S, D), jnp.float32),
            jax.ShapeDtypeStruct((S, E), jnp.float32),
            jax.ShapeDtypeStruct((S // 128, 128), jnp.int32),
            jax.ShapeDtypeStruct((S // 128, 128), jnp.int32),
            jax.ShapeDtypeStruct((S // CHUNK, CHUNK * 16), jnp.float32),
            jax.ShapeDtypeStruct((S // CHUNK, CHUNK * 16), jnp.float32),
            jax.ShapeDtypeStruct((8, 128), jnp.int32),
        ],
    )(x2, g2, b2, wgt)


def _gmm_body(s_ref, xs_ref, w1_ref, w2_ref, y_ref, w2b_scr):
    i = pl.program_id(0)
    valid = s_ref[2, i] == 1
    im1 = jnp.maximum(i - 1, 0)
    new_e = jnp.logical_or(i == 0, s_ref[0, i] != s_ref[0, im1])

    @pl.when(jnp.logical_and(valid, new_e))
    def _():
        w2b_scr[...] = w2_ref[0].astype(jnp.bfloat16)

    @pl.when(valid)
    def _():
        xb = xs_ref[...].astype(jnp.bfloat16)
        h = lax.dot_general(xb, w1_ref[0], (((1,), (1,)), ((), ())),
                            preferred_element_type=jnp.float32)
        xp = h[:, :FF]
        gt = h[:, FF:]
        og = xp * (0.5 * gt * (1.0 + lax.erf(gt * _SQRT1_2)))
        y_ref[...] = lax.dot_general(og.astype(jnp.bfloat16), w2b_scr[...],
                                     (((1,), (1,)), ((), ())),
                                     preferred_element_type=jnp.float32)


def _gmm(sched, xs, w1t, w2t):
    grid_spec = pltpu.PrefetchScalarGridSpec(
        num_scalar_prefetch=1,
        grid=(TMAX,),
        in_specs=[
            pl.BlockSpec((TILE, D), lambda i, s: (s[1, i], 0)),
            pl.BlockSpec((1, 2 * FF, D), lambda i, s: (s[0, i], 0, 0)),
            pl.BlockSpec((1, D, FF), lambda i, s: (s[0, i], 0, 0)),
        ],
        out_specs=pl.BlockSpec((TILE, D), lambda i, s: (s[1, i], 0)),
        scratch_shapes=[pltpu.VMEM((D, FF), jnp.bfloat16)],
    )
    return pl.pallas_call(
        _gmm_body,
        grid_spec=grid_spec,
        out_shape=jax.ShapeDtypeStruct((P, D), jnp.float32),
        compiler_params=pltpu.CompilerParams(
            dimension_semantics=("arbitrary",)),
    )(sched, xs, w1t, w2t)


def _sc_scatter(xn, pos1r, pos2r):
    mesh = plsc.VectorSubcoreMesh(core_axis_name="c", subcore_axis_name="s")

    @functools.partial(
        pl.kernel, mesh=mesh,
        out_type=jax.ShapeDtypeStruct((P, D), jnp.float32),
        scratch_types=[
            pltpu.VMEM((TPW,), jnp.int32),
            pltpu.VMEM((TPW,), jnp.int32),
            pltpu.VMEM((TPW, D), jnp.float32),
            pltpu.SemaphoreType.DMA,
            pltpu.SemaphoreType.DMA,
        ],
    )
    def k(xn_hbm, p1_hbm, p2_hbm, xs_hbm, i1_v, i2_v, rows_v, sem1, sem2):
        wid = lax.axis_index("s") * 2 + lax.axis_index("c")
        base = wid * TPW
        row = wid // 2
        lane = (wid % 2) * TPW
        pltpu.sync_copy(p1_hbm.at[row, pl.ds(lane, TPW)], i1_v)
        pltpu.sync_copy(p2_hbm.at[row, pl.ds(lane, TPW)], i2_v)
        pltpu.sync_copy(xn_hbm.at[pl.ds(base, TPW)], rows_v)
        c1 = pltpu.async_copy(rows_v, xs_hbm.at[i1_v], sem1)
        c2 = pltpu.async_copy(rows_v, xs_hbm.at[i2_v], sem2)
        c1.wait()
        c2.wait()

    return k(xn, pos1r, pos2r)


def _sc_combine(x2, ys, p1c, p2c, t1c, t2c):
    mesh = plsc.VectorSubcoreMesh(core_axis_name="c", subcore_axis_name="s")
    nchunk = TPW // CHUNK

    @functools.partial(
        pl.kernel, mesh=mesh,
        out_type=jax.ShapeDtypeStruct((S, D), jnp.float32),
        scratch_types=[
            pltpu.VMEM((CHUNK,), jnp.int32),
            pltpu.VMEM((CHUNK,), jnp.int32),
            pltpu.VMEM((CHUNK * 16,), jnp.float32),
            pltpu.VMEM((CHUNK * 16,), jnp.float32),
            pltpu.VMEM((CHUNK, D), jnp.float32),
            pltpu.VMEM((CHUNK, D), jnp.float32),
            pltpu.VMEM((CHUNK, D), jnp.float32),
            pltpu.SemaphoreType.DMA,
            pltpu.SemaphoreType.DMA,
        ],
    )
    def k(x_hbm, ys_hbm, p1_hbm, p2_hbm, t1_hbm, t2_hbm, out_hbm,
          i1_v, i2_v, t1_v, t2_v, acc_v, g1_v, g2_v, sem1, sem2):
        wid = lax.axis_index("s") * 2 + lax.axis_index("c")
        for c in range(nchunk):
            base = wid * TPW + c * CHUNK
            prow = wid // 2
            plane = (wid % 2) * TPW + c * CHUNK
            pltpu.sync_copy(p1_hbm.at[prow, pl.ds(plane, CHUNK)], i1_v)
            pltpu.sync_copy(p2_hbm.at[prow, pl.ds(plane, CHUNK)], i2_v)
            pltpu.sync_copy(t1_hbm.at[wid * nchunk + c], t1_v)
            pltpu.sync_copy(t2_hbm.at[wid * nchunk + c], t2_v)
            cp1 = pltpu.async_copy(ys_hbm.at[i1_v], g1_v, sem1)
            cp2 = pltpu.async_copy(ys_hbm.at[i2_v], g2_v, sem2)
            pltpu.sync_copy(x_hbm.at[pl.ds(base, CHUNK)], acc_v)
            cp1.wait()
            cp2.wait()

            def body(r, carry):
                t1s = t1_v[pl.ds(r * 16, 16)]
                t2s = t2_v[pl.ds(r * 16, 16)]
                for g in range(D // 16):
                    sl = pl.ds(g * 16, 16)
                    acc_v[r, sl] = (acc_v[r, sl]
                                    + t1s * g1_v[r, sl] + t2s * g2_v[r, sl])
                return carry

            lax.fori_loop(0, CHUNK, body, 0)
            pltpu.sync_copy(acc_v, out_hbm.at[pl.ds(base, CHUNK)])

    return k(x2, ys, p1c, p2c, t1c, t2c)


def kernel(x, gamma, beta, Wg, W1, W2):
    x2 = x.reshape(S, D)
    g2 = gamma.reshape(1, D)
    b2 = beta.reshape(1, D)
    wgt = Wg.T

    w1b = W1.astype(jnp.bfloat16)

    xn, mask, pos1, pos2, t1b, t2b, sched = _router(x2, g2, b2, wgt)
    xs = _sc_scatter(xn, pos1, pos2)
    ys = _gmm(sched, xs, w1b, W2)
    out2 = _sc_combine(x2, ys, pos1, pos2, t1b, t2b)
    return out2.reshape(1, S, D), mask.reshape(1, S, E)


# speedup vs baseline: 2.9773x; 1.0784x over previous
import functools

import jax
import jax.numpy as jnp
from jax import lax
from jax.experimental import pallas as pl
from jax.experimental.pallas import tpu as pltpu
from jax.experimental.pallas import tpu_sc as plsc

S, D, E, FF = 2048, 1024, 8, 2048
TILE = 256
TMAX = 24
P = TMAX * TILE
NW = 32
TPW = S // NW
CHUNK = 32

_SQRT1_2 = 0.7071067811865476


def _router_body(x_ref, g_ref, b_ref, wg_ref,
                 xn_ref, mask_ref, pos1_ref, pos2_ref, t1b_ref, t2b_ref,
                 sched_ref):
    x = x_ref[...]
    mu = jnp.mean(x, axis=1, keepdims=True)
    xc = x - mu
    var = jnp.mean(xc * xc, axis=1, keepdims=True)
    xn = xc * lax.rsqrt(var + 1e-5) * g_ref[...] + b_ref[...]
    xn_ref[...] = xn

    logits = jnp.dot(xn, wg_ref[...], preferred_element_type=jnp.float32)
    m = jnp.max(logits, axis=1, keepdims=True)
    ex = jnp.exp(logits - m)
    w = ex / jnp.sum(ex, axis=1, keepdims=True)

    ie = lax.broadcasted_iota(jnp.int32, (S, E), 1)
    m1 = jnp.max(w, axis=1, keepdims=True)
    i1 = jnp.min(jnp.where(w == m1, ie, E), axis=1, keepdims=True)
    wm = jnp.where(ie == i1, -1.0, w)
    m2 = jnp.max(wm, axis=1, keepdims=True)
    i2 = jnp.min(jnp.where(wm == m2, ie, E), axis=1, keepdims=True)
    ssum = m1 + m2
    t1 = m1 / ssum
    t2 = m2 / ssum
    oh1 = ie == i1
    oh2 = ie == i2
    mask_ref[...] = jnp.where(oh1, t1, 0.0) + jnp.where(oh2, t2, 0.0)
    nrows = S // CHUNK
    t1b_ref[...] = jnp.broadcast_to(
        t1.reshape(nrows, CHUNK)[:, :, None], (nrows, CHUNK, 16)
    ).reshape(nrows, CHUNK * 16)
    t2b_ref[...] = jnp.broadcast_to(
        t2.reshape(nrows, CHUNK)[:, :, None], (nrows, CHUNK, 16)
    ).reshape(nrows, CHUNK * 16)

    cnt = (oh1 | oh2).astype(jnp.float32)
    csum = cnt
    k = 1
    while k < S:
        csum = csum + jnp.concatenate(
            [jnp.zeros((k, E), jnp.float32), csum[: S - k, :]], axis=0)
        k *= 2
    counts = csum[S - 1: S, :]
    rank = (csum - cnt).astype(jnp.int32)

    eye = (lax.broadcasted_iota(jnp.int32, (E, E), 0)
           == lax.broadcasted_iota(jnp.int32, (E, E), 1))
    countsT = jnp.sum(jnp.where(eye, jnp.broadcast_to(counts, (E, E)), 0.0),
                      axis=1, keepdims=True)
    pcT = ((countsT.astype(jnp.int32) + (TILE - 1)) // TILE) * TILE
    inc = pcT
    k = 1
    while k < E:
        inc = inc + jnp.concatenate(
            [jnp.zeros((k, 1), jnp.int32), inc[: E - k, :]], axis=0)
        k *= 2
    startT = inc - pcT
    start = jnp.sum(jnp.where(eye, jnp.broadcast_to(startT, (E, E)), 0),
                    axis=0, keepdims=True)

    posm = start + rank
    pos1 = jnp.sum(jnp.where(oh1, posm, 0), axis=1, keepdims=True)
    pos2 = jnp.sum(jnp.where(oh2, posm, 0), axis=1, keepdims=True)
    pos1_ref[...] = pos1.reshape(S // 128, 128)
    pos2_ref[...] = pos2.reshape(S // 128, 128)

    it = lax.broadcasted_iota(jnp.int32, (1, 128), 1)
    base = it * TILE
    total = jnp.sum(pcT, axis=0, keepdims=True)
    nvalid = total // TILE
    valid = base < total
    base8 = jnp.broadcast_to(base, (E, 128))
    inb = (base8 >= startT) & (base8 < startT + pcT)
    e_of = jnp.sum(jnp.where(inb, lax.broadcasted_iota(jnp.int32, (E, 128), 0), 0),
                   axis=0, keepdims=True)
    last = nvalid - 1
    r_eff = jnp.where(valid, it, last)
    e_last = jnp.sum(jnp.where(it == last, e_of, 0), axis=1, keepdims=True)
    e_eff = jnp.where(valid, e_of, e_last)
    sched_ref[...] = jnp.concatenate(
        [e_eff, r_eff, valid.astype(jnp.int32)]
        + [jnp.zeros((1, 128), jnp.int32)] * 5, axis=0)


def _router(x2, g2, b2, wgt):
    return pl.pallas_call(
        _router_body,
        out_shape=[
            jax.ShapeDtypeStruct((S, D), jnp.float32),
            jax.ShapeDtypeStruct((S, E), jnp.float32),
            jax.ShapeDtypeStruct((S // 128, 128), jnp.int32),
            jax.ShapeDtypeStruct((S // 128, 128), jnp.int32),
            jax.ShapeDtypeStruct((S // CHUNK, CHUNK * 16), jnp.float32),
            jax.ShapeDtypeStruct((S // CHUNK, CHUNK * 16), jnp.float32),
            jax.ShapeDtypeStruct((8, 128), jnp.int32),
        ],
    )(x2, g2, b2, wgt)


def _gmm_body(s_ref, xs_ref, w1_ref, w2_ref, y_ref, w1b_scr):
    i = pl.program_id(0)
    valid = s_ref[2, i] == 1
    im1 = jnp.maximum(i - 1, 0)
    new_e = jnp.logical_or(i == 0, s_ref[0, i] != s_ref[0, im1])

    @pl.when(jnp.logical_and(valid, new_e))
    def _():
        w1b_scr[...] = w1_ref[0].astype(jnp.bfloat16)

    @pl.when(valid)
    def _():
        xb = xs_ref[...].astype(jnp.bfloat16)
        h = lax.dot_general(xb, w1b_scr[...], (((1,), (1,)), ((), ())),
                            preferred_element_type=jnp.float32)
        xp = h[:, :FF]
        gt = h[:, FF:]
        og = xp * (0.5 * gt * (1.0 + lax.erf(gt * _SQRT1_2)))
        y_ref[...] = lax.dot_general(og.astype(jnp.bfloat16), w2_ref[0],
                                     (((1,), (1,)), ((), ())),
                                     preferred_element_type=jnp.float32)


def _gmm(sched, xs, w1t, w2t):
    grid_spec = pltpu.PrefetchScalarGridSpec(
        num_scalar_prefetch=1,
        grid=(TMAX,),
        in_specs=[
            pl.BlockSpec((TILE, D), lambda i, s: (s[1, i], 0)),
            pl.BlockSpec((1, 2 * FF, D), lambda i, s: (s[0, i], 0, 0)),
            pl.BlockSpec((1, D, FF), lambda i, s: (s[0, i], 0, 0)),
        ],
        out_specs=pl.BlockSpec((TILE, D), lambda i, s: (s[1, i], 0)),
        scratch_shapes=[pltpu.VMEM((2 * FF, D), jnp.bfloat16)],
    )
    return pl.pallas_call(
        _gmm_body,
        grid_spec=grid_spec,
        out_shape=jax.ShapeDtypeStruct((P, D), jnp.float32),
        compiler_params=pltpu.CompilerParams(
            dimension_semantics=("arbitrary",)),
    )(sched, xs, w1t, w2t)


def _sc_scatter(xn, pos1r, pos2r):
    mesh = plsc.VectorSubcoreMesh(core_axis_name="c", subcore_axis_name="s")

    @functools.partial(
        pl.kernel, mesh=mesh,
        out_type=jax.ShapeDtypeStruct((P, D), jnp.float32),
        scratch_types=[
            pltpu.VMEM((TPW,), jnp.int32),
            pltpu.VMEM((TPW,), jnp.int32),
            pltpu.VMEM((TPW, D), jnp.float32),
            pltpu.SemaphoreType.DMA,
            pltpu.SemaphoreType.DMA,
        ],
    )
    def k(xn_hbm, p1_hbm, p2_hbm, xs_hbm, i1_v, i2_v, rows_v, sem1, sem2):
        wid = lax.axis_index("s") * 2 + lax.axis_index("c")
        base = wid * TPW
        row = wid // 2
        lane = (wid % 2) * TPW
        pltpu.sync_copy(p1_hbm.at[row, pl.ds(lane, TPW)], i1_v)
        pltpu.sync_copy(p2_hbm.at[row, pl.ds(lane, TPW)], i2_v)
        pltpu.sync_copy(xn_hbm.at[pl.ds(base, TPW)], rows_v)
        c1 = pltpu.async_copy(rows_v, xs_hbm.at[i1_v], sem1)
        c2 = pltpu.async_copy(rows_v, xs_hbm.at[i2_v], sem2)
        c1.wait()
        c2.wait()

    return k(xn, pos1r, pos2r)


def _sc_combine(x2, ys, p1c, p2c, t1c, t2c):
    mesh = plsc.VectorSubcoreMesh(core_axis_name="c", subcore_axis_name="s")
    nchunk = TPW // CHUNK

    @functools.partial(
        pl.kernel, mesh=mesh,
        out_type=jax.ShapeDtypeStruct((S, D), jnp.float32),
        scratch_types=[
            pltpu.VMEM((CHUNK,), jnp.int32),
            pltpu.VMEM((CHUNK,), jnp.int32),
            pltpu.VMEM((CHUNK * 16,), jnp.float32),
            pltpu.VMEM((CHUNK * 16,), jnp.float32),
            pltpu.VMEM((CHUNK, D), jnp.float32),
            pltpu.VMEM((CHUNK, D), jnp.float32),
            pltpu.VMEM((CHUNK, D), jnp.float32),
            pltpu.SemaphoreType.DMA,
            pltpu.SemaphoreType.DMA,
        ],
    )
    def k(x_hbm, ys_hbm, p1_hbm, p2_hbm, t1_hbm, t2_hbm, out_hbm,
          i1_v, i2_v, t1_v, t2_v, acc_v, g1_v, g2_v, sem1, sem2):
        wid = lax.axis_index("s") * 2 + lax.axis_index("c")
        for c in range(nchunk):
            base = wid * TPW + c * CHUNK
            prow = wid // 2
            plane = (wid % 2) * TPW + c * CHUNK
            pltpu.sync_copy(p1_hbm.at[prow, pl.ds(plane, CHUNK)], i1_v)
            pltpu.sync_copy(p2_hbm.at[prow, pl.ds(plane, CHUNK)], i2_v)
            pltpu.sync_copy(t1_hbm.at[wid * nchunk + c], t1_v)
            pltpu.sync_copy(t2_hbm.at[wid * nchunk + c], t2_v)
            cp1 = pltpu.async_copy(ys_hbm.at[i1_v], g1_v, sem1)
            cp2 = pltpu.async_copy(ys_hbm.at[i2_v], g2_v, sem2)
            pltpu.sync_copy(x_hbm.at[pl.ds(base, CHUNK)], acc_v)
            cp1.wait()
            cp2.wait()

            def body(r, carry):
                t1s = t1_v[pl.ds(r * 16, 16)]
                t2s = t2_v[pl.ds(r * 16, 16)]
                for g in range(D // 16):
                    sl = pl.ds(g * 16, 16)
                    acc_v[r, sl] = (acc_v[r, sl]
                                    + t1s * g1_v[r, sl] + t2s * g2_v[r, sl])
                return carry

            lax.fori_loop(0, CHUNK, body, 0)
            pltpu.sync_copy(acc_v, out_hbm.at[pl.ds(base, CHUNK)])

    return k(x2, ys, p1c, p2c, t1c, t2c)


def kernel(x, gamma, beta, Wg, W1, W2):
    x2 = x.reshape(S, D)
    g2 = gamma.reshape(1, D)
    b2 = beta.reshape(1, D)
    wgt = Wg.T

    w2b = W2.astype(jnp.bfloat16)

    xn, mask, pos1, pos2, t1b, t2b, sched = _router(x2, g2, b2, wgt)
    xs = _sc_scatter(xn, pos1, pos2)
    ys = _gmm(sched, xs, W1, w2b)
    out2 = _sc_combine(x2, ys, pos1, pos2, t1b, t2b)
    return out2.reshape(1, S, D), mask.reshape(1, S, E)


# speedup vs baseline: 3.0256x; 1.0162x over previous
import functools

import jax
import jax.numpy as jnp
from jax import lax
from jax.experimental import pallas as pl
from jax.experimental.pallas import tpu as pltpu
from jax.experimental.pallas import tpu_sc as plsc

S, D, E, FF = 2048, 1024, 8, 2048
TILE = 256
TMAX = 24
P = TMAX * TILE
NW = 32
TPW = S // NW
CHUNK = 16

_SQRT1_2 = 0.7071067811865476


def _router_body(x_ref, g_ref, b_ref, wg_ref,
                 xn_ref, mask_ref, pos1_ref, pos2_ref, t1b_ref, t2b_ref,
                 sched_ref):
    x = x_ref[...]
    mu = jnp.mean(x, axis=1, keepdims=True)
    xc = x - mu
    var = jnp.mean(xc * xc, axis=1, keepdims=True)
    xn = xc * lax.rsqrt(var + 1e-5) * g_ref[...] + b_ref[...]
    xn_ref[...] = xn

    logits = jnp.dot(xn, wg_ref[...], preferred_element_type=jnp.float32)
    m = jnp.max(logits, axis=1, keepdims=True)
    ex = jnp.exp(logits - m)
    w = ex / jnp.sum(ex, axis=1, keepdims=True)

    ie = lax.broadcasted_iota(jnp.int32, (S, E), 1)
    m1 = jnp.max(w, axis=1, keepdims=True)
    i1 = jnp.min(jnp.where(w == m1, ie, E), axis=1, keepdims=True)
    wm = jnp.where(ie == i1, -1.0, w)
    m2 = jnp.max(wm, axis=1, keepdims=True)
    i2 = jnp.min(jnp.where(wm == m2, ie, E), axis=1, keepdims=True)
    ssum = m1 + m2
    t1 = m1 / ssum
    t2 = m2 / ssum
    oh1 = ie == i1
    oh2 = ie == i2
    mask_ref[...] = jnp.where(oh1, t1, 0.0) + jnp.where(oh2, t2, 0.0)
    nrows = S // CHUNK
    t1b_ref[...] = jnp.broadcast_to(
        t1.reshape(nrows, CHUNK)[:, :, None], (nrows, CHUNK, 16)
    ).reshape(nrows, CHUNK * 16)
    t2b_ref[...] = jnp.broadcast_to(
        t2.reshape(nrows, CHUNK)[:, :, None], (nrows, CHUNK, 16)
    ).reshape(nrows, CHUNK * 16)

    cnt = (oh1 | oh2).astype(jnp.float32)
    csum = cnt
    k = 1
    while k < S:
        csum = csum + jnp.concatenate(
            [jnp.zeros((k, E), jnp.float32), csum[: S - k, :]], axis=0)
        k *= 2
    counts = csum[S - 1: S, :]
    rank = (csum - cnt).astype(jnp.int32)

    eye = (lax.broadcasted_iota(jnp.int32, (E, E), 0)
           == lax.broadcasted_iota(jnp.int32, (E, E), 1))
    countsT = jnp.sum(jnp.where(eye, jnp.broadcast_to(counts, (E, E)), 0.0),
                      axis=1, keepdims=True)
    pcT = ((countsT.astype(jnp.int32) + (TILE - 1)) // TILE) * TILE
    inc = pcT
    k = 1
    while k < E:
        inc = inc + jnp.concatenate(
            [jnp.zeros((k, 1), jnp.int32), inc[: E - k, :]], axis=0)
        k *= 2
    startT = inc - pcT
    start = jnp.sum(jnp.where(eye, jnp.broadcast_to(startT, (E, E)), 0),
                    axis=0, keepdims=True)

    posm = start + rank
    pos1 = jnp.sum(jnp.where(oh1, posm, 0), axis=1, keepdims=True)
    pos2 = jnp.sum(jnp.where(oh2, posm, 0), axis=1, keepdims=True)
    pos1_ref[...] = pos1.reshape(S // 128, 128)
    pos2_ref[...] = pos2.reshape(S // 128, 128)

    it = lax.broadcasted_iota(jnp.int32, (1, 128), 1)
    base = it * TILE
    total = jnp.sum(pcT, axis=0, keepdims=True)
    nvalid = total // TILE
    valid = base < total
    base8 = jnp.broadcast_to(base, (E, 128))
    inb = (base8 >= startT) & (base8 < startT + pcT)
    e_of = jnp.sum(jnp.where(inb, lax.broadcasted_iota(jnp.int32, (E, 128), 0), 0),
                   axis=0, keepdims=True)
    last = nvalid - 1
    r_eff = jnp.where(valid, it, last)
    e_last = jnp.sum(jnp.where(it == last, e_of, 0), axis=1, keepdims=True)
    e_eff = jnp.where(valid, e_of, e_last)
    sched_ref[...] = jnp.concatenate(
        [e_eff, r_eff, valid.astype(jnp.int32)]
        + [jnp.zeros((1, 128), jnp.int32)] * 5, axis=0)


def _router(x2, g2, b2, wgt):
    return pl.pallas_call(
        _router_body,
        out_shape=[
            jax.ShapeDtypeStruct((S, D), jnp.float32),
            jax.ShapeDtypeStruct((S, E), jnp.float32),
            jax.ShapeDtypeStruct((S // 128, 128), jnp.int32),
            jax.ShapeDtypeStruct((S // 128, 128), jnp.int32),
            jax.ShapeDtypeStruct((S // CHUNK, CHUNK * 16), jnp.float32),
            jax.ShapeDtypeStruct((S // CHUNK, CHUNK * 16), jnp.float32),
            jax.ShapeDtypeStruct((8, 128), jnp.int32),
        ],
    )(x2, g2, b2, wgt)


def _gmm_body(s_ref, xs_ref, w1_ref, w2_ref, y_ref, w1b_scr):
    i = pl.program_id(0)
    valid = s_ref[2, i] == 1
    im1 = jnp.maximum(i - 1, 0)
    new_e = jnp.logical_or(i == 0, s_ref[0, i] != s_ref[0, im1])

    @pl.when(jnp.logical_and(valid, new_e))
    def _():
        w1b_scr[...] = w1_ref[0].astype(jnp.bfloat16)

    @pl.when(valid)
    def _():
        xb = xs_ref[...].astype(jnp.bfloat16)
        h = lax.dot_general(xb, w1b_scr[...], (((1,), (1,)), ((), ())),
                            preferred_element_type=jnp.float32)
        xp = h[:, :FF]
        gt = h[:, FF:]
        og = xp * (0.5 * gt * (1.0 + lax.erf(gt * _SQRT1_2)))
        y_ref[...] = lax.dot_general(og.astype(jnp.bfloat16), w2_ref[0],
                                     (((1,), (1,)), ((), ())),
                                     preferred_element_type=jnp.float32)


def _gmm(sched, xs, w1t, w2t):
    grid_spec = pltpu.PrefetchScalarGridSpec(
        num_scalar_prefetch=1,
        grid=(TMAX,),
        in_specs=[
            pl.BlockSpec((TILE, D), lambda i, s: (s[1, i], 0)),
            pl.BlockSpec((1, 2 * FF, D), lambda i, s: (s[0, i], 0, 0)),
            pl.BlockSpec((1, D, FF), lambda i, s: (s[0, i], 0, 0)),
        ],
        out_specs=pl.BlockSpec((TILE, D), lambda i, s: (s[1, i], 0)),
        scratch_shapes=[pltpu.VMEM((2 * FF, D), jnp.bfloat16)],
    )
    return pl.pallas_call(
        _gmm_body,
        grid_spec=grid_spec,
        out_shape=jax.ShapeDtypeStruct((P, D), jnp.float32),
        compiler_params=pltpu.CompilerParams(
            dimension_semantics=("arbitrary",)),
    )(sched, xs, w1t, w2t)


def _sc_scatter(xn, pos1r, pos2r):
    mesh = plsc.VectorSubcoreMesh(core_axis_name="c", subcore_axis_name="s")

    @functools.partial(
        pl.kernel, mesh=mesh,
        out_type=jax.ShapeDtypeStruct((P, D), jnp.float32),
        scratch_types=[
            pltpu.VMEM((TPW // 2,), jnp.int32),
            pltpu.VMEM((TPW // 2,), jnp.int32),
            pltpu.VMEM((TPW // 2,), jnp.int32),
            pltpu.VMEM((TPW // 2,), jnp.int32),
            pltpu.VMEM((TPW // 2, D), jnp.float32),
            pltpu.VMEM((TPW // 2, D), jnp.float32),
            pltpu.SemaphoreType.DMA,
            pltpu.SemaphoreType.DMA,
            pltpu.SemaphoreType.DMA,
            pltpu.SemaphoreType.DMA,
            pltpu.SemaphoreType.DMA,
        ],
    )
    def k(xn_hbm, p1_hbm, p2_hbm, xs_hbm, i1a_v, i2a_v, i1b_v, i2b_v,
          ra_v, rb_v, sem0, sem1, sem2, sem3, sem4):
        H = TPW // 2
        wid = lax.axis_index("s") * 2 + lax.axis_index("c")
        base = wid * TPW
        row = wid // 2
        lane = (wid % 2) * TPW
        ca = pltpu.async_copy(xn_hbm.at[pl.ds(base, H)], ra_v, sem0)
        pltpu.sync_copy(p1_hbm.at[row, pl.ds(lane, H)], i1a_v)
        pltpu.sync_copy(p2_hbm.at[row, pl.ds(lane, H)], i2a_v)
        pltpu.sync_copy(p1_hbm.at[row, pl.ds(lane + H, H)], i1b_v)
        pltpu.sync_copy(p2_hbm.at[row, pl.ds(lane + H, H)], i2b_v)
        ca.wait()
        cb = pltpu.async_copy(xn_hbm.at[pl.ds(base + H, H)], rb_v, sem0)
        c1 = pltpu.async_copy(ra_v, xs_hbm.at[i1a_v], sem1)
        c2 = pltpu.async_copy(ra_v, xs_hbm.at[i2a_v], sem2)
        cb.wait()
        c3 = pltpu.async_copy(rb_v, xs_hbm.at[i1b_v], sem3)
        c4 = pltpu.async_copy(rb_v, xs_hbm.at[i2b_v], sem4)
        c1.wait()
        c2.wait()
        c3.wait()
        c4.wait()

    return k(xn, pos1r, pos2r)


def _sc_combine(x2, ys, p1c, p2c, t1c, t2c):
    mesh = plsc.VectorSubcoreMesh(core_axis_name="c", subcore_axis_name="s")
    nchunk = TPW // CHUNK

    buf_types = [
        pltpu.VMEM((CHUNK,), jnp.int32),
        pltpu.VMEM((CHUNK,), jnp.int32),
        pltpu.VMEM((CHUNK * 16,), jnp.float32),
        pltpu.VMEM((CHUNK * 16,), jnp.float32),
        pltpu.VMEM((CHUNK, D), jnp.float32),
        pltpu.VMEM((CHUNK, D), jnp.float32),
        pltpu.VMEM((CHUNK, D), jnp.float32),
        pltpu.SemaphoreType.DMA,
        pltpu.SemaphoreType.DMA,
        pltpu.SemaphoreType.DMA,
        pltpu.SemaphoreType.DMA,
    ]

    @functools.partial(
        pl.kernel, mesh=mesh,
        out_type=jax.ShapeDtypeStruct((S, D), jnp.float32),
        scratch_types=buf_types + buf_types,
    )
    def k(x_hbm, ys_hbm, p1_hbm, p2_hbm, t1_hbm, t2_hbm, out_hbm, *scr):
        bufs = (scr[:11], scr[11:])
        wid = lax.axis_index("s") * 2 + lax.axis_index("c")
        base0 = wid * TPW
        prow = wid // 2

        def issue(c, buf):
            i1_v, i2_v, t1_v, t2_v, g1_v, g2_v, acc_v, sg1, sg2, sx, _ = buf
            plane = (wid % 2) * TPW + c * CHUNK
            pltpu.sync_copy(p1_hbm.at[prow, pl.ds(plane, CHUNK)], i1_v)
            pltpu.sync_copy(p2_hbm.at[prow, pl.ds(plane, CHUNK)], i2_v)
            pltpu.sync_copy(t1_hbm.at[wid * nchunk + c], t1_v)
            pltpu.sync_copy(t2_hbm.at[wid * nchunk + c], t2_v)
            return (pltpu.async_copy(ys_hbm.at[i1_v], g1_v, sg1),
                    pltpu.async_copy(ys_hbm.at[i2_v], g2_v, sg2),
                    pltpu.async_copy(x_hbm.at[pl.ds(base0 + c * CHUNK, CHUNK)],
                                     acc_v, sx))

        pend = issue(0, bufs[0])
        outc = [None, None]
        for c in range(nchunk):
            buf = bufs[c % 2]
            i1_v, i2_v, t1_v, t2_v, g1_v, g2_v, acc_v, sg1, sg2, sx, so = buf
            if c + 1 < nchunk:
                nbuf = bufs[(c + 1) % 2]
                if outc[(c + 1) % 2] is not None:
                    outc[(c + 1) % 2].wait()
                pend_next = issue(c + 1, nbuf)
            for h in pend:
                h.wait()

            def body(r, carry):
                t1s = t1_v[pl.ds(r * 16, 16)]
                t2s = t2_v[pl.ds(r * 16, 16)]
                for g in range(D // 16):
                    sl = pl.ds(g * 16, 16)
                    acc_v[r, sl] = (acc_v[r, sl]
                                    + t1s * g1_v[r, sl] + t2s * g2_v[r, sl])
                return carry

            lax.fori_loop(0, CHUNK, body, 0)
            outc[c % 2] = pltpu.async_copy(
                acc_v, out_hbm.at[pl.ds(base0 + c * CHUNK, CHUNK)], so)
            if c + 1 < nchunk:
                pend = pend_next
        for h in outc:
            if h is not None:
                h.wait()

    return k(x2, ys, p1c, p2c, t1c, t2c)


def kernel(x, gamma, beta, Wg, W1, W2):
    x2 = x.reshape(S, D)
    g2 = gamma.reshape(1, D)
    b2 = beta.reshape(1, D)
    wgt = Wg.T

    w2b = W2.astype(jnp.bfloat16)

    xn, mask, pos1, pos2, t1b, t2b, sched = _router(x2, g2, b2, wgt)
    xs = _sc_scatter(xn, pos1, pos2)
    ys = _gmm(sched, xs, W1, w2b)
    out2 = _sc_combine(x2, ys, pos1, pos2, t1b, t2b)
    return out2.reshape(1, S, D), mask.reshape(1, S, E)


# speedup vs baseline: 3.0476x; 1.0073x over previous
import functools

import jax
import jax.numpy as jnp
from jax import lax
from jax.experimental import pallas as pl
from jax.experimental.pallas import tpu as pltpu
from jax.experimental.pallas import tpu_sc as plsc

S, D, E, FF = 2048, 1024, 8, 2048
TILE = 256
TMAX = 24
P = TMAX * TILE
NW = 32
TPW = S // NW
CHUNK = 16

_SQRT1_2 = 0.7071067811865476


def _router_body(x_ref, g_ref, b_ref, wg_ref,
                 xn_ref, mask_ref, pos1_ref, pos2_ref, t1b_ref, t2b_ref,
                 sched_ref):
    x = x_ref[...]
    mu = jnp.mean(x, axis=1, keepdims=True)
    xc = x - mu
    var = jnp.mean(xc * xc, axis=1, keepdims=True)
    xn = xc * lax.rsqrt(var + 1e-5) * g_ref[...] + b_ref[...]
    xn_ref[...] = xn

    logits = lax.dot_general(xn, wg_ref[...], (((1,), (1,)), ((), ())),
                             preferred_element_type=jnp.float32)
    m = jnp.max(logits, axis=1, keepdims=True)
    ex = jnp.exp(logits - m)
    w = ex / jnp.sum(ex, axis=1, keepdims=True)

    ie = lax.broadcasted_iota(jnp.int32, (S, E), 1)
    m1 = jnp.max(w, axis=1, keepdims=True)
    i1 = jnp.min(jnp.where(w == m1, ie, E), axis=1, keepdims=True)
    wm = jnp.where(ie == i1, -1.0, w)
    m2 = jnp.max(wm, axis=1, keepdims=True)
    i2 = jnp.min(jnp.where(wm == m2, ie, E), axis=1, keepdims=True)
    ssum = m1 + m2
    t1 = m1 / ssum
    t2 = m2 / ssum
    oh1 = ie == i1
    oh2 = ie == i2
    mask_ref[...] = jnp.where(oh1, t1, 0.0) + jnp.where(oh2, t2, 0.0)
    nrows = S // CHUNK
    t1b_ref[...] = jnp.broadcast_to(
        t1.reshape(nrows, CHUNK)[:, :, None], (nrows, CHUNK, 16)
    ).reshape(nrows, CHUNK * 16)
    t2b_ref[...] = jnp.broadcast_to(
        t2.reshape(nrows, CHUNK)[:, :, None], (nrows, CHUNK, 16)
    ).reshape(nrows, CHUNK * 16)

    cnt = (oh1 | oh2).astype(jnp.float32)
    csum = cnt
    k = 1
    while k < S:
        csum = csum + jnp.concatenate(
            [jnp.zeros((k, E), jnp.float32), csum[: S - k, :]], axis=0)
        k *= 2
    counts = csum[S - 1: S, :]
    rank = (csum - cnt).astype(jnp.int32)

    eye = (lax.broadcasted_iota(jnp.int32, (E, E), 0)
           == lax.broadcasted_iota(jnp.int32, (E, E), 1))
    countsT = jnp.sum(jnp.where(eye, jnp.broadcast_to(counts, (E, E)), 0.0),
                      axis=1, keepdims=True)
    pcT = ((countsT.astype(jnp.int32) + (TILE - 1)) // TILE) * TILE
    inc = pcT
    k = 1
    while k < E:
        inc = inc + jnp.concatenate(
            [jnp.zeros((k, 1), jnp.int32), inc[: E - k, :]], axis=0)
        k *= 2
    startT = inc - pcT
    start = jnp.sum(jnp.where(eye, jnp.broadcast_to(startT, (E, E)), 0),
                    axis=0, keepdims=True)

    posm = start + rank
    pos1 = jnp.sum(jnp.where(oh1, posm, 0), axis=1, keepdims=True)
    pos2 = jnp.sum(jnp.where(oh2, posm, 0), axis=1, keepdims=True)
    pos1_ref[...] = pos1.reshape(S // 128, 128)
    pos2_ref[...] = pos2.reshape(S // 128, 128)

    it = lax.broadcasted_iota(jnp.int32, (1, 128), 1)
    base = it * TILE
    total = jnp.sum(pcT, axis=0, keepdims=True)
    nvalid = total // TILE
    valid = base < total
    base8 = jnp.broadcast_to(base, (E, 128))
    inb = (base8 >= startT) & (base8 < startT + pcT)
    e_of = jnp.sum(jnp.where(inb, lax.broadcasted_iota(jnp.int32, (E, 128), 0), 0),
                   axis=0, keepdims=True)
    last = nvalid - 1
    r_eff = jnp.where(valid, it, last)
    e_last = jnp.sum(jnp.where(it == last, e_of, 0), axis=1, keepdims=True)
    e_eff = jnp.where(valid, e_of, e_last)
    sched_ref[...] = jnp.concatenate(
        [e_eff, r_eff, valid.astype(jnp.int32)]
        + [jnp.zeros((1, 128), jnp.int32)] * 5, axis=0)


def _router(x2, g2, b2, wgt):
    return pl.pallas_call(
        _router_body,
        out_shape=[
            jax.ShapeDtypeStruct((S, D), jnp.float32),
            jax.ShapeDtypeStruct((S, E), jnp.float32),
            jax.ShapeDtypeStruct((S // 128, 128), jnp.int32),
            jax.ShapeDtypeStruct((S // 128, 128), jnp.int32),
            jax.ShapeDtypeStruct((S // CHUNK, CHUNK * 16), jnp.float32),
            jax.ShapeDtypeStruct((S // CHUNK, CHUNK * 16), jnp.float32),
            jax.ShapeDtypeStruct((8, 128), jnp.int32),
        ],
    )(x2, g2, b2, wgt)


def _gmm_body(s_ref, xs_ref, w1_ref, w2_ref, y_ref, w1b_scr):
    i = pl.program_id(0)
    valid = s_ref[2, i] == 1
    im1 = jnp.maximum(i - 1, 0)
    new_e = jnp.logical_or(i == 0, s_ref[0, i] != s_ref[0, im1])

    @pl.when(jnp.logical_and(valid, new_e))
    def _():
        w1b_scr[...] = w1_ref[0].astype(jnp.bfloat16)

    @pl.when(valid)
    def _():
        xb = xs_ref[...].astype(jnp.bfloat16)
        h = lax.dot_general(xb, w1b_scr[...], (((1,), (1,)), ((), ())),
                            preferred_element_type=jnp.float32)
        xp = h[:, :FF]
        gt = h[:, FF:]
        og = xp * (0.5 * gt * (1.0 + lax.erf(gt * _SQRT1_2)))
        y_ref[...] = lax.dot_general(og.astype(jnp.bfloat16), w2_ref[0],
                                     (((1,), (1,)), ((), ())),
                                     preferred_element_type=jnp.float32)


def _gmm(sched, xs, w1t, w2t):
    grid_spec = pltpu.PrefetchScalarGridSpec(
        num_scalar_prefetch=1,
        grid=(TMAX,),
        in_specs=[
            pl.BlockSpec((TILE, D), lambda i, s: (s[1, i], 0)),
            pl.BlockSpec((1, 2 * FF, D), lambda i, s: (s[0, i], 0, 0)),
            pl.BlockSpec((1, D, FF), lambda i, s: (s[0, i], 0, 0)),
        ],
        out_specs=pl.BlockSpec((TILE, D), lambda i, s: (s[1, i], 0)),
        scratch_shapes=[pltpu.VMEM((2 * FF, D), jnp.bfloat16)],
    )
    return pl.pallas_call(
        _gmm_body,
        grid_spec=grid_spec,
        out_shape=jax.ShapeDtypeStruct((P, D), jnp.float32),
        compiler_params=pltpu.CompilerParams(
            dimension_semantics=("arbitrary",)),
    )(sched, xs, w1t, w2t)


def _sc_scatter(xn, pos1r, pos2r):
    mesh = plsc.VectorSubcoreMesh(core_axis_name="c", subcore_axis_name="s")

    @functools.partial(
        pl.kernel, mesh=mesh,
        out_type=jax.ShapeDtypeStruct((P, D), jnp.float32),
        scratch_types=[
            pltpu.VMEM((TPW // 2,), jnp.int32),
            pltpu.VMEM((TPW // 2,), jnp.int32),
            pltpu.VMEM((TPW // 2,), jnp.int32),
            pltpu.VMEM((TPW // 2,), jnp.int32),
            pltpu.VMEM((TPW // 2, D), jnp.float32),
            pltpu.VMEM((TPW // 2, D), jnp.float32),
            pltpu.SemaphoreType.DMA,
            pltpu.SemaphoreType.DMA,
            pltpu.SemaphoreType.DMA,
            pltpu.SemaphoreType.DMA,
            pltpu.SemaphoreType.DMA,
        ],
    )
    def k(xn_hbm, p1_hbm, p2_hbm, xs_hbm, i1a_v, i2a_v, i1b_v, i2b_v,
          ra_v, rb_v, sem0, sem1, sem2, sem3, sem4):
        H = TPW // 2
        wid = lax.axis_index("s") * 2 + lax.axis_index("c")
        base = wid * TPW
        row = wid // 2
        lane = (wid % 2) * TPW
        ca = pltpu.async_copy(xn_hbm.at[pl.ds(base, H)], ra_v, sem0)
        pltpu.sync_copy(p1_hbm.at[row, pl.ds(lane, H)], i1a_v)
        pltpu.sync_copy(p2_hbm.at[row, pl.ds(lane, H)], i2a_v)
        pltpu.sync_copy(p1_hbm.at[row, pl.ds(lane + H, H)], i1b_v)
        pltpu.sync_copy(p2_hbm.at[row, pl.ds(lane + H, H)], i2b_v)
        ca.wait()
        cb = pltpu.async_copy(xn_hbm.at[pl.ds(base + H, H)], rb_v, sem0)
        c1 = pltpu.async_copy(ra_v, xs_hbm.at[i1a_v], sem1)
        c2 = pltpu.async_copy(ra_v, xs_hbm.at[i2a_v], sem2)
        cb.wait()
        c3 = pltpu.async_copy(rb_v, xs_hbm.at[i1b_v], sem3)
        c4 = pltpu.async_copy(rb_v, xs_hbm.at[i2b_v], sem4)
        c1.wait()
        c2.wait()
        c3.wait()
        c4.wait()

    return k(xn, pos1r, pos2r)


def _sc_combine(x2, ys, p1c, p2c, t1c, t2c):
    mesh = plsc.VectorSubcoreMesh(core_axis_name="c", subcore_axis_name="s")
    nchunk = TPW // CHUNK

    buf_types = [
        pltpu.VMEM((CHUNK,), jnp.int32),
        pltpu.VMEM((CHUNK,), jnp.int32),
        pltpu.VMEM((CHUNK * 16,), jnp.float32),
        pltpu.VMEM((CHUNK * 16,), jnp.float32),
        pltpu.VMEM((CHUNK, D), jnp.float32),
        pltpu.VMEM((CHUNK, D), jnp.float32),
        pltpu.VMEM((CHUNK, D), jnp.float32),
        pltpu.SemaphoreType.DMA,
        pltpu.SemaphoreType.DMA,
        pltpu.SemaphoreType.DMA,
        pltpu.SemaphoreType.DMA,
    ]

    @functools.partial(
        pl.kernel, mesh=mesh,
        out_type=jax.ShapeDtypeStruct((S, D), jnp.float32),
        scratch_types=buf_types + buf_types,
    )
    def k(x_hbm, ys_hbm, p1_hbm, p2_hbm, t1_hbm, t2_hbm, out_hbm, *scr):
        bufs = (scr[:11], scr[11:])
        wid = lax.axis_index("s") * 2 + lax.axis_index("c")
        base0 = wid * TPW
        prow = wid // 2

        def issue(c, buf):
            i1_v, i2_v, t1_v, t2_v, g1_v, g2_v, acc_v, sg1, sg2, sx, _ = buf
            plane = (wid % 2) * TPW + c * CHUNK
            pltpu.sync_copy(p1_hbm.at[prow, pl.ds(plane, CHUNK)], i1_v)
            pltpu.sync_copy(p2_hbm.at[prow, pl.ds(plane, CHUNK)], i2_v)
            pltpu.sync_copy(t1_hbm.at[wid * nchunk + c], t1_v)
            pltpu.sync_copy(t2_hbm.at[wid * nchunk + c], t2_v)
            return (pltpu.async_copy(ys_hbm.at[i1_v], g1_v, sg1),
                    pltpu.async_copy(ys_hbm.at[i2_v], g2_v, sg2),
                    pltpu.async_copy(x_hbm.at[pl.ds(base0 + c * CHUNK, CHUNK)],
                                     acc_v, sx))

        pend = issue(0, bufs[0])
        outc = [None, None]
        for c in range(nchunk):
            buf = bufs[c % 2]
            i1_v, i2_v, t1_v, t2_v, g1_v, g2_v, acc_v, sg1, sg2, sx, so = buf
            if c + 1 < nchunk:
                nbuf = bufs[(c + 1) % 2]
                if outc[(c + 1) % 2] is not None:
                    outc[(c + 1) % 2].wait()
                pend_next = issue(c + 1, nbuf)
            for h in pend:
                h.wait()

            def body(r, carry):
                t1s = t1_v[pl.ds(r * 16, 16)]
                t2s = t2_v[pl.ds(r * 16, 16)]
                for g in range(D // 16):
                    sl = pl.ds(g * 16, 16)
                    acc_v[r, sl] = (acc_v[r, sl]
                                    + t1s * g1_v[r, sl] + t2s * g2_v[r, sl])
                return carry

            lax.fori_loop(0, CHUNK, body, 0)
            outc[c % 2] = pltpu.async_copy(
                acc_v, out_hbm.at[pl.ds(base0 + c * CHUNK, CHUNK)], so)
            if c + 1 < nchunk:
                pend = pend_next
        for h in outc:
            if h is not None:
                h.wait()

    return k(x2, ys, p1c, p2c, t1c, t2c)


def kernel(x, gamma, beta, Wg, W1, W2):
    x2 = x.reshape(S, D)
    g2 = gamma.reshape(1, D)
    b2 = beta.reshape(1, D)

    w2b = W2.astype(jnp.bfloat16)

    xn, mask, pos1, pos2, t1b, t2b, sched = _router(x2, g2, b2, Wg)
    xs = _sc_scatter(xn, pos1, pos2)
    ys = _gmm(sched, xs, W1, w2b)
    out2 = _sc_combine(x2, ys, pos1, pos2, t1b, t2b)
    return out2.reshape(1, S, D), mask.reshape(1, S, E)
